# baseline (speedup 1.0000x reference)
import functools

import jax
import jax.numpy as jnp
from jax import lax
from jax.experimental import pallas as pl
from jax.experimental.pallas import tpu as pltpu
from jax.experimental.pallas import tpu_sc as plsc

VOCAB = 100000
EMBED = 128
BATCH = 1024

_NC = 2
_NS = 16
_NW = _NC * _NS
_B_PER_W = BATCH // _NW

TILE_V = 3072
NV = pl.cdiv(VOCAB, TILE_V)
REM_V = VOCAB - (NV - 1) * TILE_V
NBUF = 4


def _gather_body(table_hbm, idx_hbm, out_hbm, idx_v, rows_v, sem):
    wid = lax.axis_index("s") * _NC + lax.axis_index("c")
    base = wid * _B_PER_W
    pltpu.sync_copy(idx_hbm.at[pl.ds(base, _B_PER_W)], idx_v)
    pltpu.async_copy(table_hbm.at[idx_v], rows_v, sem).wait()
    pltpu.sync_copy(rows_v, out_hbm.at[pl.ds(base, _B_PER_W)])


@functools.lru_cache(maxsize=1)
def _sc_gather():
    return pl.kernel(
        _gather_body,
        out_type=jax.ShapeDtypeStruct((BATCH, EMBED), jnp.float32),
        mesh=plsc.VectorSubcoreMesh(core_axis_name="c", subcore_axis_name="s"),
        scratch_types=[
            pltpu.VMEM((_B_PER_W,), jnp.int32),
            pltpu.VMEM((_B_PER_W, EMBED), jnp.float32),
            pltpu.SemaphoreType.DMA,
        ],
    )


def _row_start(j):
    return j * TILE_V


def _proj_body(w_ref, x_ref, b_ref, o_hbm, acc, sems):
    j = pl.program_id(0)
    slot = lax.rem(j, NBUF)

    @pl.when(j >= NBUF)
    def _wait_prev():
        pltpu.make_async_copy(
            acc.at[slot],
            o_hbm.at[pl.ds((j - NBUF) * TILE_V, TILE_V), :],
            sems.at[slot],
        ).wait()

    y = lax.dot_general(
        w_ref[...],
        x_ref[...],
        dimension_numbers=(((1,), (1,)), ((), ())),
        preferred_element_type=jnp.float32,
    )
    acc[slot] = y + lax.transpose(b_ref[...], (1, 0))

    @pl.when(j < NV - 1)
    def _start_full():
        pltpu.make_async_copy(
            acc.at[slot],
            o_hbm.at[pl.ds(j * TILE_V, TILE_V), :],
            sems.at[slot],
        ).start()

    @pl.when(j == NV - 1)
    def _start_rem_and_drain():
        pltpu.make_async_copy(
            acc.at[slot, : REM_V, :],
            o_hbm.at[pl.ds(j * TILE_V, REM_V), :],
            sems.at[slot],
        ).start()
        for k in range(NBUF - 1, 0, -1):
            jj = j - k
            slot_k = lax.rem(jj, NBUF)
            pltpu.make_async_copy(
                acc.at[slot_k],
                o_hbm.at[pl.ds(jj * TILE_V, TILE_V), :],
                sems.at[slot_k],
            ).wait()
        pltpu.make_async_copy(
            acc.at[slot, : REM_V, :],
            o_hbm.at[pl.ds(j * TILE_V, REM_V), :],
            sems.at[slot],
        ).wait()


def _project_t(x, lin_w, b_row):
    return pl.pallas_call(
        _proj_body,
        grid=(NV,),
        in_specs=[
            pl.BlockSpec((TILE_V, EMBED), lambda j: (j, 0)),
            pl.BlockSpec((BATCH, EMBED), lambda j: (0, 0)),
            pl.BlockSpec((1, TILE_V), lambda j: (0, j)),
        ],
        out_specs=pl.BlockSpec(memory_space=pl.ANY),
        out_shape=jax.ShapeDtypeStruct((VOCAB, BATCH), jnp.float32),
        scratch_shapes=[
            pltpu.VMEM((NBUF, TILE_V, BATCH), jnp.float32),
            pltpu.SemaphoreType.DMA((NBUF,)),
        ],
    )(lin_w, x, b_row)


def kernel(center_word, emb_table, lin_w, lin_b):
    x = _sc_gather()(emb_table, center_word)
    out_t = _project_t(x, lin_w, lin_b.reshape(1, VOCAB))
    return out_t.T


# speedup vs baseline: 1.0100x; 1.0100x over previous
import functools

import jax
import jax.numpy as jnp
from jax import lax
from jax.experimental import pallas as pl
from jax.experimental.pallas import tpu as pltpu
from jax.experimental.pallas import tpu_sc as plsc

VOCAB = 100000
EMBED = 128
BATCH = 1024

_NC = 1
_NS = 16
_NW = _NC * _NS
_B_PER_W = BATCH // _NW

TILE_V = 4096
NV = pl.cdiv(VOCAB, TILE_V)
REM_V = VOCAB - (NV - 1) * TILE_V
NBUF = 3


def _gather_body(table_hbm, idx_hbm, out_hbm, idx_v, rows_v, sem):
    wid = lax.axis_index("s") * _NC + lax.axis_index("c")
    base = wid * _B_PER_W
    pltpu.sync_copy(idx_hbm.at[pl.ds(base, _B_PER_W)], idx_v)
    pltpu.async_copy(table_hbm.at[idx_v], rows_v, sem).wait()
    pltpu.sync_copy(rows_v, out_hbm.at[pl.ds(base, _B_PER_W)])


@functools.lru_cache(maxsize=1)
def _sc_gather():
    return pl.kernel(
        _gather_body,
        out_type=jax.ShapeDtypeStruct((BATCH, EMBED), jnp.float32),
        mesh=plsc.VectorSubcoreMesh(core_axis_name="c", subcore_axis_name="s", num_cores=1),
        scratch_types=[
            pltpu.VMEM((_B_PER_W,), jnp.int32),
            pltpu.VMEM((_B_PER_W, EMBED), jnp.float32),
            pltpu.SemaphoreType.DMA,
        ],
    )


def _row_start(j):
    return j * TILE_V


def _proj_body(w_ref, x_ref, b_ref, o_hbm, acc, sems):
    j = pl.program_id(0)
    slot = lax.rem(j, NBUF)

    @pl.when(j >= NBUF)
    def _wait_prev():
        pltpu.make_async_copy(
            acc.at[slot],
            o_hbm.at[pl.ds((j - NBUF) * TILE_V, TILE_V), :],
            sems.at[slot],
        ).wait()

    y = lax.dot_general(
        w_ref[...],
        x_ref[...],
        dimension_numbers=(((1,), (1,)), ((), ())),
        preferred_element_type=jnp.float32,
    )
    acc[slot] = y + lax.transpose(b_ref[...], (1, 0))

    @pl.when(j < NV - 1)
    def _start_full():
        pltpu.make_async_copy(
            acc.at[slot],
            o_hbm.at[pl.ds(j * TILE_V, TILE_V), :],
            sems.at[slot],
        ).start()

    @pl.when(j == NV - 1)
    def _start_rem_and_drain():
        pltpu.make_async_copy(
            acc.at[slot, : REM_V, :],
            o_hbm.at[pl.ds(j * TILE_V, REM_V), :],
            sems.at[slot],
        ).start()
        for k in range(NBUF - 1, 0, -1):
            jj = j - k
            slot_k = lax.rem(jj, NBUF)
            pltpu.make_async_copy(
                acc.at[slot_k],
                o_hbm.at[pl.ds(jj * TILE_V, TILE_V), :],
                sems.at[slot_k],
            ).wait()
        pltpu.make_async_copy(
            acc.at[slot, : REM_V, :],
            o_hbm.at[pl.ds(j * TILE_V, REM_V), :],
            sems.at[slot],
        ).wait()


def _project_t(x, lin_w, b_row):
    return pl.pallas_call(
        _proj_body,
        grid=(NV,),
        in_specs=[
            pl.BlockSpec((TILE_V, EMBED), lambda j: (j, 0)),
            pl.BlockSpec((BATCH, EMBED), lambda j: (0, 0)),
            pl.BlockSpec((1, TILE_V), lambda j: (0, j)),
        ],
        out_specs=pl.BlockSpec(memory_space=pl.ANY),
        out_shape=jax.ShapeDtypeStruct((VOCAB, BATCH), jnp.float32),
        scratch_shapes=[
            pltpu.VMEM((NBUF, TILE_V, BATCH), jnp.float32),
            pltpu.SemaphoreType.DMA((NBUF,)),
        ],
    )(lin_w, x, b_row)


def kernel(center_word, emb_table, lin_w, lin_b):
    x = _sc_gather()(emb_table, center_word)
    out_t = _project_t(x, lin_w, lin_b.reshape(1, VOCAB))
    return out_t.T


# speedup vs baseline: 1.0103x; 1.0004x over previous
import functools

import jax
import jax.numpy as jnp
from jax import lax
from jax.experimental import pallas as pl
from jax.experimental.pallas import tpu as pltpu
from jax.experimental.pallas import tpu_sc as plsc

VOCAB = 100000
EMBED = 128
BATCH = 1024

_NC = 1
_NS = 16
_NW = _NC * _NS
_B_PER_W = BATCH // _NW

TILE_V = 4096
NV = pl.cdiv(VOCAB, TILE_V)
REM_V = VOCAB - (NV - 1) * TILE_V
NBUF = 3


def _gather_body(table_hbm, idx_hbm, out_hbm, idx_v, rows_v, sem):
    wid = lax.axis_index("s") * _NC + lax.axis_index("c")
    base = wid * _B_PER_W
    pltpu.sync_copy(idx_hbm.at[pl.ds(base, _B_PER_W)], idx_v)
    pltpu.async_copy(table_hbm.at[idx_v], rows_v, sem).wait()
    pltpu.sync_copy(rows_v, out_hbm.at[pl.ds(base, _B_PER_W)])


@functools.lru_cache(maxsize=1)
def _sc_gather():
    return pl.kernel(
        _gather_body,
        out_type=jax.ShapeDtypeStruct((BATCH, EMBED), jnp.float32),
        mesh=plsc.VectorSubcoreMesh(core_axis_name="c", subcore_axis_name="s", num_cores=1),
        scratch_types=[
            pltpu.VMEM((_B_PER_W,), jnp.int32),
            pltpu.VMEM((_B_PER_W, EMBED), jnp.float32),
            pltpu.SemaphoreType.DMA,
        ],
    )


def _proj_body(w_ref, x_ref, b_ref, o_hbm, acc, sems):
    j = pl.program_id(0)
    slot = lax.rem(j, NBUF)

    @pl.when(j >= NBUF)
    def _wait_prev():
        pltpu.make_async_copy(
            acc.at[slot],
            o_hbm.at[pl.ds((j - NBUF) * TILE_V, TILE_V), :],
            sems.at[slot],
        ).wait()

    y = lax.dot_general(
        w_ref[...],
        x_ref[...],
        dimension_numbers=(((1,), (1,)), ((), ())),
        preferred_element_type=jnp.float32,
    )
    acc[slot] = y + lax.transpose(b_ref[...], (1, 0))

    @pl.when(j < NV - 1)
    def _start_full():
        pltpu.make_async_copy(
            acc.at[slot],
            o_hbm.at[pl.ds(j * TILE_V, TILE_V), :],
            sems.at[slot],
        ).start()

    @pl.when(j == NV - 1)
    def _start_rem_and_drain():
        pltpu.make_async_copy(
            acc.at[slot, : REM_V, :],
            o_hbm.at[pl.ds(j * TILE_V, REM_V), :],
            sems.at[slot],
        ).start()
        for k in range(NBUF - 1, 0, -1):
            jj = j - k
            slot_k = lax.rem(jj, NBUF)
            pltpu.make_async_copy(
                acc.at[slot_k],
                o_hbm.at[pl.ds(jj * TILE_V, TILE_V), :],
                sems.at[slot_k],
            ).wait()
        pltpu.make_async_copy(
            acc.at[slot, : REM_V, :],
            o_hbm.at[pl.ds(j * TILE_V, REM_V), :],
            sems.at[slot],
        ).wait()


def _project_t(x, lin_w, b_row):
    return pl.pallas_call(
        _proj_body,
        grid=(NV,),
        in_specs=[
            pl.BlockSpec((TILE_V, EMBED), lambda j: (j, 0)),
            pl.BlockSpec((BATCH, EMBED), lambda j: (0, 0)),
            pl.BlockSpec((1, TILE_V), lambda j: (0, j)),
        ],
        out_specs=pl.BlockSpec(memory_space=pl.ANY),
        out_shape=jax.ShapeDtypeStruct((VOCAB, BATCH), jnp.float32),
        scratch_shapes=[
            pltpu.VMEM((NBUF, TILE_V, BATCH), jnp.float32),
            pltpu.SemaphoreType.DMA((NBUF,)),
        ],
    )(lin_w, x, b_row)


def kernel(center_word, emb_table, lin_w, lin_b):
    x = _sc_gather()(emb_table, center_word)
    out_t = _project_t(x, lin_w, lin_b.reshape(1, VOCAB))
    return out_t.T
